# baseline (speedup 1.0000x reference)
import functools

import jax
import jax.numpy as jnp
from jax import lax
from jax.experimental import pallas as pl
from jax.experimental.pallas import tpu as pltpu
from jax.experimental.pallas import tpu_sc as plsc

N = 10000
D = 128
HID = 256
E = 320000
NC = 2
NS = 16
NW = NC * NS
EW = E // NW
CH = 128
NCHK = 80
EWP = NCHK * CH
NPAD = 10240
TR = NPAD // NS
BN = 400

_mesh = plsc.VectorSubcoreMesh(core_axis_name="c", subcore_axis_name="s")


@functools.partial(
    pl.kernel,
    mesh=_mesh,
    out_type=jax.ShapeDtypeStruct((NC, NPAD), jnp.float32),
    scratch_types=[
        pltpu.VMEM((NCHK, CH), jnp.int32),
        pltpu.VMEM((CH,), jnp.float32),
        pltpu.VMEM((CH,), jnp.float32),
        pltpu.VMEM_SHARED((NPAD,), jnp.float32),
    ],
)
def _deg_sc(dst_hbm, out_hbm, dst_v, ones_v, zeros_v, acc):
    c = lax.axis_index("c")
    s = lax.axis_index("s")
    w = c * NS + s
    for j in range(CH // 16):
        ones_v[pl.ds(j * 16, 16)] = jnp.ones((16,), jnp.float32)
        zeros_v[pl.ds(j * 16, 16)] = jnp.zeros((16,), jnp.float32)
    for k in range(TR // CH):
        pltpu.sync_copy(zeros_v, acc.at[pl.ds(s * TR + k * CH, CH)])
    plsc.subcore_barrier()

    pltpu.sync_copy(dst_hbm.at[w], dst_v)

    def body(j, carry):
        pltpu.sync_copy(ones_v, acc.at[dst_v.at[j]], add=True)
        return carry

    lax.fori_loop(0, NCHK, body, 0)
    plsc.subcore_barrier()
    pltpu.sync_copy(acc.at[pl.ds(s * TR, TR)], out_hbm.at[c, pl.ds(s * TR, TR)])


@functools.partial(
    pl.kernel,
    mesh=_mesh,
    out_type=jax.ShapeDtypeStruct((NC, NPAD, D), jnp.float32),
    scratch_types=[
        pltpu.VMEM((NCHK, CH), jnp.int32),
        pltpu.VMEM((NCHK, CH), jnp.int32),
        pltpu.VMEM((CH, D), jnp.float32),
        pltpu.VMEM_SHARED((NPAD, D), jnp.float32),
        pltpu.SemaphoreType.DMA,
    ],
)
def _agg_sc(table_hbm, src_hbm, dst_hbm, out_hbm, src_v, dst_v,
            rows0, acc, gsem):
    c = lax.axis_index("c")
    s = lax.axis_index("s")
    w = c * NS + s

    def zrow(r, carry):
        for j in range(D // 16):
            rows0[r, pl.ds(j * 16, 16)] = jnp.zeros((16,), jnp.float32)
        return carry

    lax.fori_loop(0, CH, zrow, 0)
    for k in range(TR // CH):
        pltpu.sync_copy(rows0, acc.at[pl.ds(s * TR + k * CH, CH), :])
    plsc.subcore_barrier()

    pltpu.sync_copy(src_hbm.at[w], src_v)
    pltpu.sync_copy(dst_hbm.at[w], dst_v)

    def body(j, carry):
        pltpu.async_copy(table_hbm.at[src_v.at[j]], rows0, gsem).wait()
        pltpu.sync_copy(rows0, acc.at[dst_v.at[j]], add=True)
        return carry

    lax.fori_loop(0, NCHK, body, 0)
    plsc.subcore_barrier()
    pltpu.sync_copy(acc.at[pl.ds(s * TR, TR), :],
                    out_hbm.at[c, pl.ds(s * TR, TR), :])


def _tc1_body(deg_ref, x_ref, xd_ref, d_ref):
    degs = deg_ref[0] + deg_ref[1] + 1.0
    dv = lax.rsqrt(degs)
    d_ref[...] = dv
    xd_ref[...] = x_ref[...] * dv


def _tc1(deg2, x):
    return pl.pallas_call(
        _tc1_body,
        grid=(N // BN,),
        in_specs=[
            pl.BlockSpec((NC, BN, 1), lambda i: (0, i, 0)),
            pl.BlockSpec((BN, D), lambda i: (i, 0)),
        ],
        out_specs=[
            pl.BlockSpec((BN, D), lambda i: (i, 0)),
            pl.BlockSpec((BN, 1), lambda i: (i, 0)),
        ],
        out_shape=[
            jax.ShapeDtypeStruct((N, D), jnp.float32),
            jax.ShapeDtypeStruct((N, 1), jnp.float32),
        ],
    )(deg2, x)


def _tc2_body(acc_ref, x_ref, d_ref, w1_ref, b1_ref, w2_ref, q2_ref, q2d_ref):
    dv = d_ref[...]
    m1 = dv * (acc_ref[0] + acc_ref[1]) + (dv * dv) * x_ref[...]
    h1 = jnp.maximum(
        jnp.dot(m1, w1_ref[...], preferred_element_type=jnp.float32)
        + b1_ref[...], 0.0)
    q2 = jnp.dot(h1, w2_ref[...], preferred_element_type=jnp.float32)
    q2_ref[...] = q2
    q2d_ref[...] = q2 * dv


def _tc2(acc1, x, d, W1, b1, W2):
    return pl.pallas_call(
        _tc2_body,
        grid=(N // BN,),
        in_specs=[
            pl.BlockSpec((NC, BN, D), lambda i: (0, i, 0)),
            pl.BlockSpec((BN, D), lambda i: (i, 0)),
            pl.BlockSpec((BN, 1), lambda i: (i, 0)),
            pl.BlockSpec((D, HID), lambda i: (0, 0)),
            pl.BlockSpec((1, HID), lambda i: (0, 0)),
            pl.BlockSpec((HID, D), lambda i: (0, 0)),
        ],
        out_specs=[
            pl.BlockSpec((BN, D), lambda i: (i, 0)),
            pl.BlockSpec((BN, D), lambda i: (i, 0)),
        ],
        out_shape=[
            jax.ShapeDtypeStruct((N, D), jnp.float32),
            jax.ShapeDtypeStruct((N, D), jnp.float32),
        ],
    )(acc1, x, d, W1, b1, W2)


def _tc3_body(acc_ref, q2_ref, d_ref, b2_ref, wfc_ref, bfc_ref, out_ref):
    dv = d_ref[...]
    q2 = q2_ref[...]
    h2 = jnp.maximum(
        dv * (acc_ref[0] + acc_ref[1]) + (dv * dv) * q2 + b2_ref[...], 0.0)
    out_ref[...] = (
        jnp.dot(h2, wfc_ref[...], preferred_element_type=jnp.float32)
        + bfc_ref[...])


def _tc3(acc2, q2, d, b2, Wfc, bfc):
    return pl.pallas_call(
        _tc3_body,
        grid=(N // BN,),
        in_specs=[
            pl.BlockSpec((NC, BN, D), lambda i: (0, i, 0)),
            pl.BlockSpec((BN, D), lambda i: (i, 0)),
            pl.BlockSpec((BN, 1), lambda i: (i, 0)),
            pl.BlockSpec((1, D), lambda i: (0, 0)),
            pl.BlockSpec((D, D), lambda i: (0, 0)),
            pl.BlockSpec((1, D), lambda i: (0, 0)),
        ],
        out_specs=pl.BlockSpec((BN, D), lambda i: (i, 0)),
        out_shape=jax.ShapeDtypeStruct((N, D), jnp.float32),
    )(acc2, q2, d, b2, Wfc, bfc)


def kernel(x, edge_index, W1, b1, W2, b2, Wfc, bfc):
    src = edge_index[0].reshape(NW, EW)
    dst = edge_index[1].reshape(NW, EW)
    pad = EWP - EW
    src_p = jnp.pad(src, ((0, 0), (0, pad))).reshape(NW, NCHK, CH)
    dst_p = jnp.pad(dst, ((0, 0), (0, pad)),
                    constant_values=N).reshape(NW, NCHK, CH)

    deg2 = _deg_sc(dst_p)
    xd, d = _tc1(deg2.reshape(NC, NPAD, 1), x)
    acc1 = _agg_sc(xd, src_p, dst_p)
    q2, q2d = _tc2(acc1, x, d, W1, b1.reshape(1, HID), W2)
    acc2 = _agg_sc(q2d, src_p, dst_p)
    return _tc3(acc2, q2, d, b2.reshape(1, D), Wfc, bfc.reshape(1, D))


# speedup vs baseline: 1.4393x; 1.4393x over previous
import functools

import jax
import jax.numpy as jnp
from jax import lax
from jax.experimental import pallas as pl
from jax.experimental.pallas import tpu as pltpu
from jax.experimental.pallas import tpu_sc as plsc

N = 10000
D = 128
HID = 256
E = 320000
NC = 2
NS = 16
NW = NC * NS
EW = E // NW
CH = 128
NCHK = 79
EWP = NCHK * CH
NPAD = 10240
TR = NPAD // NS
BN = 400

_mesh = plsc.VectorSubcoreMesh(core_axis_name="c", subcore_axis_name="s")


@functools.partial(
    pl.kernel,
    mesh=_mesh,
    out_type=jax.ShapeDtypeStruct((NC, NPAD), jnp.float32),
    scratch_types=[
        pltpu.VMEM((NCHK, CH), jnp.int32),
        pltpu.VMEM((CH,), jnp.float32),
        pltpu.VMEM((CH,), jnp.float32),
        pltpu.VMEM_SHARED((NPAD,), jnp.float32),
    ],
)
def _deg_sc(dst_hbm, out_hbm, dst_v, ones_v, zeros_v, acc):
    c = lax.axis_index("c")
    s = lax.axis_index("s")
    w = c * NS + s
    for j in range(CH // 16):
        ones_v[pl.ds(j * 16, 16)] = jnp.ones((16,), jnp.float32)
        zeros_v[pl.ds(j * 16, 16)] = jnp.zeros((16,), jnp.float32)
    for k in range(TR // CH):
        pltpu.sync_copy(zeros_v, acc.at[pl.ds(s * TR + k * CH, CH)])
    plsc.subcore_barrier()

    pltpu.sync_copy(dst_hbm.at[w], dst_v)

    def body(j, carry):
        pltpu.sync_copy(ones_v, acc.at[dst_v.at[j]], add=True)
        return carry

    lax.fori_loop(0, NCHK, body, 0)
    plsc.subcore_barrier()
    pltpu.sync_copy(acc.at[pl.ds(s * TR, TR)], out_hbm.at[c, pl.ds(s * TR, TR)])


@functools.partial(
    pl.kernel,
    mesh=_mesh,
    out_type=jax.ShapeDtypeStruct((NC, NPAD, D), jnp.float32),
    scratch_types=[
        pltpu.VMEM((NCHK, CH), jnp.int32),
        pltpu.VMEM((NCHK, CH), jnp.int32),
        pltpu.VMEM((CH, D), jnp.float32),
        pltpu.VMEM_SHARED((NPAD, D), jnp.float32),
        pltpu.SemaphoreType.DMA,
    ],
)
def _agg_sc(table_hbm, src_hbm, dst_hbm, out_hbm, src_v, dst_v,
            rows0, acc, gsem):
    c = lax.axis_index("c")
    s = lax.axis_index("s")
    w = c * NS + s

    def zrow(r, carry):
        for j in range(D // 16):
            rows0[r, pl.ds(j * 16, 16)] = jnp.zeros((16,), jnp.float32)
        return carry

    lax.fori_loop(0, CH, zrow, 0)
    for k in range(TR // CH):
        pltpu.sync_copy(rows0, acc.at[pl.ds(s * TR + k * CH, CH), :])
    plsc.subcore_barrier()

    pltpu.sync_copy(src_hbm.at[w], src_v)
    pltpu.sync_copy(dst_hbm.at[w], dst_v)

    def body(j, carry):
        pltpu.async_copy(table_hbm.at[src_v.at[j]], rows0, gsem).wait()
        pltpu.sync_copy(rows0, acc.at[dst_v.at[j]], add=True)
        return carry

    lax.fori_loop(0, NCHK, body, 0)
    plsc.subcore_barrier()
    pltpu.sync_copy(acc.at[pl.ds(s * TR, TR), :],
                    out_hbm.at[c, pl.ds(s * TR, TR), :])


def _tc1_body(deg_ref, x_ref, xd_ref, d_ref):
    degs = deg_ref[0] + deg_ref[1] + 1.0
    dv = lax.rsqrt(degs)
    d_ref[...] = dv
    xd_ref[...] = x_ref[...] * dv


def _tc1(deg2, x):
    return pl.pallas_call(
        _tc1_body,
        grid=(N // BN,),
        in_specs=[
            pl.BlockSpec((NC, BN, 1), lambda i: (0, i, 0)),
            pl.BlockSpec((BN, D), lambda i: (i, 0)),
        ],
        out_specs=[
            pl.BlockSpec((BN, D), lambda i: (i, 0)),
            pl.BlockSpec((BN, 1), lambda i: (i, 0)),
        ],
        out_shape=[
            jax.ShapeDtypeStruct((N, D), jnp.float32),
            jax.ShapeDtypeStruct((N, 1), jnp.float32),
        ],
    )(deg2, x)


def _tc2_body(acc_ref, x_ref, d_ref, w1_ref, b1_ref, w2_ref, q2_ref, q2d_ref):
    dv = d_ref[...]
    m1 = dv * (acc_ref[0] + acc_ref[1]) + (dv * dv) * x_ref[...]
    h1 = jnp.maximum(
        jnp.dot(m1, w1_ref[...], preferred_element_type=jnp.float32)
        + b1_ref[...], 0.0)
    q2 = jnp.dot(h1, w2_ref[...], preferred_element_type=jnp.float32)
    q2_ref[...] = q2
    q2d_ref[...] = q2 * dv


def _tc2(acc1, x, d, W1, b1, W2):
    return pl.pallas_call(
        _tc2_body,
        grid=(N // BN,),
        in_specs=[
            pl.BlockSpec((NC, BN, D), lambda i: (0, i, 0)),
            pl.BlockSpec((BN, D), lambda i: (i, 0)),
            pl.BlockSpec((BN, 1), lambda i: (i, 0)),
            pl.BlockSpec((D, HID), lambda i: (0, 0)),
            pl.BlockSpec((1, HID), lambda i: (0, 0)),
            pl.BlockSpec((HID, D), lambda i: (0, 0)),
        ],
        out_specs=[
            pl.BlockSpec((BN, D), lambda i: (i, 0)),
            pl.BlockSpec((BN, D), lambda i: (i, 0)),
        ],
        out_shape=[
            jax.ShapeDtypeStruct((N, D), jnp.float32),
            jax.ShapeDtypeStruct((N, D), jnp.float32),
        ],
    )(acc1, x, d, W1, b1, W2)


def _tc3_body(acc_ref, q2_ref, d_ref, b2_ref, wfc_ref, bfc_ref, out_ref):
    dv = d_ref[...]
    q2 = q2_ref[...]
    h2 = jnp.maximum(
        dv * (acc_ref[0] + acc_ref[1]) + (dv * dv) * q2 + b2_ref[...], 0.0)
    out_ref[...] = (
        jnp.dot(h2, wfc_ref[...], preferred_element_type=jnp.float32)
        + bfc_ref[...])


def _tc3(acc2, q2, d, b2, Wfc, bfc):
    return pl.pallas_call(
        _tc3_body,
        grid=(N // BN,),
        in_specs=[
            pl.BlockSpec((NC, BN, D), lambda i: (0, i, 0)),
            pl.BlockSpec((BN, D), lambda i: (i, 0)),
            pl.BlockSpec((BN, 1), lambda i: (i, 0)),
            pl.BlockSpec((1, D), lambda i: (0, 0)),
            pl.BlockSpec((D, D), lambda i: (0, 0)),
            pl.BlockSpec((1, D), lambda i: (0, 0)),
        ],
        out_specs=pl.BlockSpec((BN, D), lambda i: (i, 0)),
        out_shape=jax.ShapeDtypeStruct((N, D), jnp.float32),
    )(acc2, q2, d, b2, Wfc, bfc)


def kernel(x, edge_index, W1, b1, W2, b2, Wfc, bfc):
    src = edge_index[0].reshape(NW, EW)
    dst = edge_index[1].reshape(NW, EW)
    pad = EWP - EW
    src_p = jnp.pad(src, ((0, 0), (0, pad))).reshape(NW, NCHK, CH)
    dst_p = jnp.pad(dst, ((0, 0), (0, pad)),
                    constant_values=N).reshape(NW, NCHK, CH)

    deg2 = _deg_sc(dst_p)
    xd, d = _tc1(deg2.reshape(NC, NPAD, 1), x)
    acc1 = _agg_sc(xd, src_p, dst_p)
    q2, q2d = _tc2(acc1, x, d, W1, b1.reshape(1, HID), W2)
    acc2 = _agg_sc(q2d, src_p, dst_p)
    return _tc3(acc2, q2, d, b2.reshape(1, D), Wfc, bfc.reshape(1, D))


# speedup vs baseline: 1.4423x; 1.0021x over previous
import functools

import jax
import jax.numpy as jnp
from jax import lax
from jax.experimental import pallas as pl
from jax.experimental.pallas import tpu as pltpu
from jax.experimental.pallas import tpu_sc as plsc

N = 10000
D = 128
HID = 256
E = 320000
NC = 2
NS = 16
NW = NC * NS
EW = E // NW
CH = 128
NCHK = 79
EWP = NCHK * CH
NPAD = 10240
TR = NPAD // NS
BN = 400

_mesh = plsc.VectorSubcoreMesh(core_axis_name="c", subcore_axis_name="s")


@functools.partial(
    pl.kernel,
    mesh=_mesh,
    out_type=jax.ShapeDtypeStruct((NC, NPAD), jnp.float32),
    scratch_types=[
        pltpu.VMEM((NCHK, CH), jnp.int32),
        pltpu.VMEM((CH,), jnp.float32),
        pltpu.VMEM((CH,), jnp.float32),
        pltpu.VMEM_SHARED((NPAD,), jnp.float32),
    ],
)
def _deg_sc(dst_hbm, out_hbm, dst_v, ones_v, zeros_v, acc):
    c = lax.axis_index("c")
    s = lax.axis_index("s")
    w = c * NS + s
    for j in range(CH // 16):
        ones_v[pl.ds(j * 16, 16)] = jnp.ones((16,), jnp.float32)
        zeros_v[pl.ds(j * 16, 16)] = jnp.zeros((16,), jnp.float32)
    for k in range(TR // CH):
        pltpu.sync_copy(zeros_v, acc.at[pl.ds(s * TR + k * CH, CH)])
    plsc.subcore_barrier()

    pltpu.sync_copy(dst_hbm.at[w], dst_v)

    def body(j, carry):
        pltpu.sync_copy(ones_v, acc.at[dst_v.at[j]], add=True)
        return carry

    lax.fori_loop(0, NCHK, body, 0)
    plsc.subcore_barrier()
    pltpu.sync_copy(acc.at[pl.ds(s * TR, TR)], out_hbm.at[c, pl.ds(s * TR, TR)])


@functools.partial(
    pl.kernel,
    mesh=_mesh,
    out_type=jax.ShapeDtypeStruct((NC, NPAD, D), jnp.float32),
    scratch_types=[
        pltpu.VMEM((NCHK, CH), jnp.int32),
        pltpu.VMEM((NCHK, CH), jnp.int32),
        pltpu.VMEM((CH, D), jnp.float32),
        pltpu.VMEM_SHARED((NPAD, D), jnp.float32),
        pltpu.SemaphoreType.DMA,
    ],
)
def _agg_sc(table_hbm, src_hbm, dst_hbm, out_hbm, src_v, dst_v,
            rows0, acc, gsem):
    c = lax.axis_index("c")
    s = lax.axis_index("s")
    w = c * NS + s

    def zrow(r, carry):
        for j in range(D // 16):
            rows0[r, pl.ds(j * 16, 16)] = jnp.zeros((16,), jnp.float32)
        return carry

    lax.fori_loop(0, CH, zrow, 0)
    for k in range(TR // CH):
        pltpu.sync_copy(rows0, acc.at[pl.ds(s * TR + k * CH, CH), :])
    plsc.subcore_barrier()

    pltpu.sync_copy(src_hbm.at[w], src_v)
    pltpu.sync_copy(dst_hbm.at[w], dst_v)

    def body(j, carry):
        pltpu.async_copy(table_hbm.at[src_v.at[j]], rows0, gsem).wait()
        pltpu.sync_copy(rows0, acc.at[dst_v.at[j]], add=True)
        return carry

    lax.fori_loop(0, NCHK, body, 0)
    plsc.subcore_barrier()
    pltpu.sync_copy(acc.at[pl.ds(s * TR, TR), :],
                    out_hbm.at[c, pl.ds(s * TR, TR), :])


def _tc1_body(deg_ref, x_ref, xd_ref, d_ref):
    degs = deg_ref[0] + deg_ref[1] + 1.0
    dv = lax.rsqrt(degs)
    d_ref[...] = dv
    xd_ref[...] = x_ref[...] * dv


def _tc1(deg2, x):
    return pl.pallas_call(
        _tc1_body,
        grid=(N // BN,),
        in_specs=[
            pl.BlockSpec((NC, BN, 1), lambda i: (0, i, 0)),
            pl.BlockSpec((BN, D), lambda i: (i, 0)),
        ],
        out_specs=[
            pl.BlockSpec((BN, D), lambda i: (i, 0)),
            pl.BlockSpec((BN, 1), lambda i: (i, 0)),
        ],
        out_shape=[
            jax.ShapeDtypeStruct((N, D), jnp.float32),
            jax.ShapeDtypeStruct((N, 1), jnp.float32),
        ],
    )(deg2, x)


def _tc2_body(acc_ref, x_ref, d_ref, w1_ref, b1_ref, w2_ref, q2_ref, q2d_ref):
    dv = d_ref[...]
    m1 = dv * (acc_ref[0] + acc_ref[1]) + (dv * dv) * x_ref[...]
    h1 = jnp.maximum(
        jnp.dot(m1, w1_ref[...], preferred_element_type=jnp.float32)
        + b1_ref[...], 0.0)
    q2 = jnp.dot(h1, w2_ref[...], preferred_element_type=jnp.float32)
    q2_ref[...] = q2
    q2d_ref[...] = q2 * dv


def _tc2(acc1, x, d, W1, b1, W2):
    return pl.pallas_call(
        _tc2_body,
        grid=(N // BN,),
        in_specs=[
            pl.BlockSpec((NC, BN, D), lambda i: (0, i, 0)),
            pl.BlockSpec((BN, D), lambda i: (i, 0)),
            pl.BlockSpec((BN, 1), lambda i: (i, 0)),
            pl.BlockSpec((D, HID), lambda i: (0, 0)),
            pl.BlockSpec((1, HID), lambda i: (0, 0)),
            pl.BlockSpec((HID, D), lambda i: (0, 0)),
        ],
        out_specs=[
            pl.BlockSpec((BN, D), lambda i: (i, 0)),
            pl.BlockSpec((BN, D), lambda i: (i, 0)),
        ],
        out_shape=[
            jax.ShapeDtypeStruct((N, D), jnp.float32),
            jax.ShapeDtypeStruct((N, D), jnp.float32),
        ],
    )(acc1, x, d, W1, b1, W2)


def _tc3_body(acc_ref, q2_ref, d_ref, b2_ref, wfc_ref, bfc_ref, out_ref):
    dv = d_ref[...]
    q2 = q2_ref[...]
    h2 = jnp.maximum(
        dv * (acc_ref[0] + acc_ref[1]) + (dv * dv) * q2 + b2_ref[...], 0.0)
    out_ref[...] = (
        jnp.dot(h2, wfc_ref[...], preferred_element_type=jnp.float32)
        + bfc_ref[...])


def _tc3(acc2, q2, d, b2, Wfc, bfc):
    return pl.pallas_call(
        _tc3_body,
        grid=(N // BN,),
        in_specs=[
            pl.BlockSpec((NC, BN, D), lambda i: (0, i, 0)),
            pl.BlockSpec((BN, D), lambda i: (i, 0)),
            pl.BlockSpec((BN, 1), lambda i: (i, 0)),
            pl.BlockSpec((1, D), lambda i: (0, 0)),
            pl.BlockSpec((D, D), lambda i: (0, 0)),
            pl.BlockSpec((1, D), lambda i: (0, 0)),
        ],
        out_specs=pl.BlockSpec((BN, D), lambda i: (i, 0)),
        out_shape=jax.ShapeDtypeStruct((N, D), jnp.float32),
    )(acc2, q2, d, b2, Wfc, bfc)


def kernel(x, edge_index, W1, b1, W2, b2, Wfc, bfc):
    src = edge_index[0].reshape(NW, EW)
    dst = edge_index[1].reshape(NW, EW)
    pad = EWP - EW
    src_p = jnp.pad(src, ((0, 0), (0, pad))).reshape(NW, NCHK, CH)
    junk = (jnp.arange(NW, dtype=jnp.int32)[:, None] * pad
            + jnp.arange(pad, dtype=jnp.int32)[None, :]) % (NPAD - N) + N
    dst_p = jnp.concatenate([dst, junk], axis=1).reshape(NW, NCHK, CH)

    deg2 = _deg_sc(dst_p)
    xd, d = _tc1(deg2.reshape(NC, NPAD, 1), x)
    acc1 = _agg_sc(xd, src_p, dst_p)
    q2, q2d = _tc2(acc1, x, d, W1, b1.reshape(1, HID), W2)
    acc2 = _agg_sc(q2d, src_p, dst_p)
    return _tc3(acc2, q2, d, b2.reshape(1, D), Wfc, bfc.reshape(1, D))


# speedup vs baseline: 2.2295x; 1.5458x over previous
import functools

import jax
import jax.numpy as jnp
from jax import lax
from jax.experimental import pallas as pl
from jax.experimental.pallas import tpu as pltpu
from jax.experimental.pallas import tpu_sc as plsc

N = 10000
D = 128
HID = 256
E = 320000
NC = 2
NS = 16
NW = NC * NS
EW = E // NW
CH = 128
NCHK = 79
EWP = NCHK * CH
NPAD = 10240
TR = NPAD // NS
BN = 400

_mesh = plsc.VectorSubcoreMesh(core_axis_name="c", subcore_axis_name="s")


@functools.partial(
    pl.kernel,
    mesh=_mesh,
    out_type=jax.ShapeDtypeStruct((NC, NPAD), jnp.float32),
    scratch_types=[
        pltpu.VMEM((NCHK, CH), jnp.int32),
        pltpu.VMEM((CH,), jnp.float32),
        pltpu.VMEM((CH,), jnp.float32),
        pltpu.VMEM_SHARED((NPAD,), jnp.float32),
    ],
)
def _deg_sc(dst_hbm, out_hbm, dst_v, ones_v, zeros_v, acc):
    c = lax.axis_index("c")
    s = lax.axis_index("s")
    w = c * NS + s
    for j in range(CH // 16):
        ones_v[pl.ds(j * 16, 16)] = jnp.ones((16,), jnp.float32)
        zeros_v[pl.ds(j * 16, 16)] = jnp.zeros((16,), jnp.float32)
    for k in range(TR // CH):
        pltpu.sync_copy(zeros_v, acc.at[pl.ds(s * TR + k * CH, CH)])
    plsc.subcore_barrier()

    pltpu.sync_copy(dst_hbm.at[w], dst_v)

    def body(j, carry):
        pltpu.sync_copy(ones_v, acc.at[dst_v.at[j]], add=True)
        return carry

    lax.fori_loop(0, NCHK, body, 0)
    plsc.subcore_barrier()
    pltpu.sync_copy(acc.at[pl.ds(s * TR, TR)], out_hbm.at[c, pl.ds(s * TR, TR)])


@functools.partial(
    pl.kernel,
    mesh=_mesh,
    out_type=jax.ShapeDtypeStruct((NC, NPAD, D), jnp.float32),
    scratch_types=[
        pltpu.VMEM((NCHK, CH), jnp.int32),
        pltpu.VMEM((NCHK, CH), jnp.int32),
        pltpu.VMEM((CH, D), jnp.float32),
        pltpu.VMEM_SHARED((NPAD, D), jnp.float32),
        pltpu.SemaphoreType.DMA,
    ],
)
def _agg_sc(table_hbm, src_hbm, dst_hbm, out_hbm, src_v, dst_v,
            rows0, acc, gsem):
    c = lax.axis_index("c")
    s = lax.axis_index("s")
    w = c * NS + s

    def zrow(r, carry):
        for j in range(D // 16):
            rows0[r, pl.ds(j * 16, 16)] = jnp.zeros((16,), jnp.float32)
        return carry

    lax.fori_loop(0, CH, zrow, 0)
    for k in range(TR // CH):
        pltpu.sync_copy(rows0, acc.at[pl.ds(s * TR + k * CH, CH), :])
    plsc.subcore_barrier()

    pltpu.sync_copy(src_hbm.at[w], src_v)
    pltpu.sync_copy(dst_hbm.at[w], dst_v)

    def body(j, carry):
        pltpu.async_copy(table_hbm.at[src_v.at[j]], rows0, gsem).wait()
        pltpu.sync_copy(rows0, acc.at[dst_v.at[j]], add=True)
        return carry

    lax.fori_loop(0, NCHK, body, 0)
    plsc.subcore_barrier()
    pltpu.sync_copy(acc.at[pl.ds(s * TR, TR), :],
                    out_hbm.at[c, pl.ds(s * TR, TR), :])


def _tc1_body(deg_ref, x_ref, xd_ref, d_ref):
    degs = deg_ref[0] + deg_ref[1] + 1.0
    dv = lax.rsqrt(degs)
    d_ref[...] = dv
    xd_ref[...] = x_ref[...] * dv


def _tc1(deg2, x):
    return pl.pallas_call(
        _tc1_body,
        grid=(N // BN,),
        in_specs=[
            pl.BlockSpec((NC, BN, 1), lambda i: (0, i, 0)),
            pl.BlockSpec((BN, D), lambda i: (i, 0)),
        ],
        out_specs=[
            pl.BlockSpec((BN, D), lambda i: (i, 0)),
            pl.BlockSpec((BN, 1), lambda i: (i, 0)),
        ],
        out_shape=[
            jax.ShapeDtypeStruct((N, D), jnp.float32),
            jax.ShapeDtypeStruct((N, 1), jnp.float32),
        ],
    )(deg2, x)


def _tc2_body(acc_ref, x_ref, d_ref, w1_ref, b1_ref, w2_ref, q2_ref, q2d_ref):
    dv = d_ref[...]
    m1 = dv * (acc_ref[0] + acc_ref[1]) + (dv * dv) * x_ref[...]
    h1 = jnp.maximum(
        jnp.dot(m1, w1_ref[...], preferred_element_type=jnp.float32)
        + b1_ref[...], 0.0)
    q2 = jnp.dot(h1, w2_ref[...], preferred_element_type=jnp.float32)
    q2_ref[...] = q2
    q2d_ref[...] = q2 * dv


def _tc2(acc1, x, d, W1, b1, W2):
    return pl.pallas_call(
        _tc2_body,
        grid=(N // BN,),
        in_specs=[
            pl.BlockSpec((NC, BN, D), lambda i: (0, i, 0)),
            pl.BlockSpec((BN, D), lambda i: (i, 0)),
            pl.BlockSpec((BN, 1), lambda i: (i, 0)),
            pl.BlockSpec((D, HID), lambda i: (0, 0)),
            pl.BlockSpec((1, HID), lambda i: (0, 0)),
            pl.BlockSpec((HID, D), lambda i: (0, 0)),
        ],
        out_specs=[
            pl.BlockSpec((BN, D), lambda i: (i, 0)),
            pl.BlockSpec((BN, D), lambda i: (i, 0)),
        ],
        out_shape=[
            jax.ShapeDtypeStruct((N, D), jnp.float32),
            jax.ShapeDtypeStruct((N, D), jnp.float32),
        ],
    )(acc1, x, d, W1, b1, W2)


def _tc3_body(acc_ref, q2_ref, d_ref, b2_ref, wfc_ref, bfc_ref, out_ref):
    dv = d_ref[...]
    q2 = q2_ref[...]
    h2 = jnp.maximum(
        dv * (acc_ref[0] + acc_ref[1]) + (dv * dv) * q2 + b2_ref[...], 0.0)
    out_ref[...] = (
        jnp.dot(h2, wfc_ref[...], preferred_element_type=jnp.float32)
        + bfc_ref[...])


def _tc3(acc2, q2, d, b2, Wfc, bfc):
    return pl.pallas_call(
        _tc3_body,
        grid=(N // BN,),
        in_specs=[
            pl.BlockSpec((NC, BN, D), lambda i: (0, i, 0)),
            pl.BlockSpec((BN, D), lambda i: (i, 0)),
            pl.BlockSpec((BN, 1), lambda i: (i, 0)),
            pl.BlockSpec((1, D), lambda i: (0, 0)),
            pl.BlockSpec((D, D), lambda i: (0, 0)),
            pl.BlockSpec((1, D), lambda i: (0, 0)),
        ],
        out_specs=pl.BlockSpec((BN, D), lambda i: (i, 0)),
        out_shape=jax.ShapeDtypeStruct((N, D), jnp.float32),
    )(acc2, q2, d, b2, Wfc, bfc)


def kernel(x, edge_index, W1, b1, W2, b2, Wfc, bfc):
    src = edge_index[0].reshape(NW, EW)
    dst = edge_index[1].reshape(NW, EW)
    pad = EWP - EW
    junk_s = (jnp.arange(NW, dtype=jnp.int32)[:, None] * pad
              + jnp.arange(pad, dtype=jnp.int32)[None, :] * 79) % N
    src_p = jnp.concatenate([src, junk_s], axis=1).reshape(NW, NCHK, CH)
    junk = (jnp.arange(NW, dtype=jnp.int32)[:, None] * pad
            + jnp.arange(pad, dtype=jnp.int32)[None, :]) % (NPAD - N) + N
    dst_p = jnp.concatenate([dst, junk], axis=1).reshape(NW, NCHK, CH)

    deg2 = _deg_sc(dst_p)
    xd, d = _tc1(deg2.reshape(NC, NPAD, 1), x)
    acc1 = _agg_sc(xd, src_p, dst_p)
    q2, q2d = _tc2(acc1, x, d, W1, b1.reshape(1, HID), W2)
    acc2 = _agg_sc(q2d, src_p, dst_p)
    return _tc3(acc2, q2, d, b2.reshape(1, D), Wfc, bfc.reshape(1, D))


# speedup vs baseline: 2.4642x; 1.1053x over previous
import functools

import jax
import jax.numpy as jnp
from jax import lax
from jax.experimental import pallas as pl
from jax.experimental.pallas import tpu as pltpu
from jax.experimental.pallas import tpu_sc as plsc

N = 10000
D = 128
HID = 256
E = 320000
NC = 2
NS = 16
NW = NC * NS
EW = E // NW
CH = 128
NCHK = 80
HCH = NCHK // 2
EWP = NCHK * CH
NPAD = 10240
TR = NPAD // NS
BN = 400

_mesh = plsc.VectorSubcoreMesh(core_axis_name="c", subcore_axis_name="s")


@functools.partial(
    pl.kernel,
    mesh=_mesh,
    out_type=jax.ShapeDtypeStruct((NC, NPAD), jnp.float32),
    scratch_types=[
        pltpu.VMEM((HCH, CH), jnp.int32),
        pltpu.VMEM((CH,), jnp.float32),
        pltpu.VMEM((CH,), jnp.float32),
        pltpu.VMEM_SHARED((NPAD,), jnp.float32),
    ],
)
def _deg_sc(dst_hbm, out_hbm, dst_v, ones_v, zeros_v, acc):
    c = lax.axis_index("c")
    s = lax.axis_index("s")
    w = c * NS + s
    for j in range(CH // 16):
        ones_v[pl.ds(j * 16, 16)] = jnp.ones((16,), jnp.float32)
        zeros_v[pl.ds(j * 16, 16)] = jnp.zeros((16,), jnp.float32)
    for k in range(TR // CH):
        pltpu.sync_copy(zeros_v, acc.at[pl.ds(s * TR + k * CH, CH)])
    plsc.subcore_barrier()

    def body(j, carry):
        pltpu.sync_copy(ones_v, acc.at[dst_v.at[j]], add=True)
        return carry

    for p in range(2):
        pltpu.sync_copy(dst_hbm.at[w, p], dst_v)
        lax.fori_loop(0, HCH, body, 0)
    plsc.subcore_barrier()
    pltpu.sync_copy(acc.at[pl.ds(s * TR, TR)], out_hbm.at[c, pl.ds(s * TR, TR)])


@functools.partial(
    pl.kernel,
    mesh=_mesh,
    out_type=jax.ShapeDtypeStruct((NC, NPAD, D), jnp.float32),
    scratch_types=[
        pltpu.VMEM((HCH, CH), jnp.int32),
        pltpu.VMEM((HCH, CH), jnp.int32),
        pltpu.VMEM((CH, D), jnp.float32),
        pltpu.VMEM((CH, D), jnp.float32),
        pltpu.VMEM_SHARED((NPAD, D), jnp.float32),
        pltpu.SemaphoreType.DMA,
        pltpu.SemaphoreType.DMA,
    ],
)
def _agg_sc(table_hbm, src_hbm, dst_hbm, out_hbm, src_v, dst_v,
            rows0, rows1, acc, gsem, gsem1):
    c = lax.axis_index("c")
    s = lax.axis_index("s")
    w = c * NS + s

    def zrow(r, carry):
        for j in range(D // 16):
            rows0[r, pl.ds(j * 16, 16)] = jnp.zeros((16,), jnp.float32)
        return carry

    lax.fori_loop(0, CH, zrow, 0)
    for k in range(TR // CH):
        pltpu.sync_copy(rows0, acc.at[pl.ds(s * TR + k * CH, CH), :])
    plsc.subcore_barrier()

    def body(i, carry):
        j0 = i * 2
        j1 = j0 + 1
        h0 = pltpu.async_copy(table_hbm.at[src_v.at[j0]], rows0, gsem)
        h0.wait()
        h1 = pltpu.async_copy(table_hbm.at[src_v.at[j1]], rows1, gsem1)
        pltpu.sync_copy(rows0, acc.at[dst_v.at[j0]], add=True)
        h1.wait()
        pltpu.sync_copy(rows1, acc.at[dst_v.at[j1]], add=True)
        return carry

    for p in range(2):
        pltpu.sync_copy(src_hbm.at[w, p], src_v)
        pltpu.sync_copy(dst_hbm.at[w, p], dst_v)
        lax.fori_loop(0, HCH // 2, body, 0)
    plsc.subcore_barrier()
    pltpu.sync_copy(acc.at[pl.ds(s * TR, TR), :],
                    out_hbm.at[c, pl.ds(s * TR, TR), :])


def _tc1_body(deg_ref, x_ref, xd_ref, d_ref):
    degs = deg_ref[0] + deg_ref[1] + 1.0
    dv = lax.rsqrt(degs)
    d_ref[...] = dv
    xd_ref[...] = x_ref[...] * dv


def _tc1(deg2, x):
    return pl.pallas_call(
        _tc1_body,
        grid=(N // BN,),
        in_specs=[
            pl.BlockSpec((NC, BN, 1), lambda i: (0, i, 0)),
            pl.BlockSpec((BN, D), lambda i: (i, 0)),
        ],
        out_specs=[
            pl.BlockSpec((BN, D), lambda i: (i, 0)),
            pl.BlockSpec((BN, 1), lambda i: (i, 0)),
        ],
        out_shape=[
            jax.ShapeDtypeStruct((N, D), jnp.float32),
            jax.ShapeDtypeStruct((N, 1), jnp.float32),
        ],
    )(deg2, x)


def _tc2_body(acc_ref, x_ref, d_ref, w1_ref, b1_ref, w2_ref, q2_ref, q2d_ref):
    dv = d_ref[...]
    m1 = dv * (acc_ref[0] + acc_ref[1]) + (dv * dv) * x_ref[...]
    h1 = jnp.maximum(
        jnp.dot(m1, w1_ref[...], preferred_element_type=jnp.float32)
        + b1_ref[...], 0.0)
    q2 = jnp.dot(h1, w2_ref[...], preferred_element_type=jnp.float32)
    q2_ref[...] = q2
    q2d_ref[...] = q2 * dv


def _tc2(acc1, x, d, W1, b1, W2):
    return pl.pallas_call(
        _tc2_body,
        grid=(N // BN,),
        in_specs=[
            pl.BlockSpec((NC, BN, D), lambda i: (0, i, 0)),
            pl.BlockSpec((BN, D), lambda i: (i, 0)),
            pl.BlockSpec((BN, 1), lambda i: (i, 0)),
            pl.BlockSpec((D, HID), lambda i: (0, 0)),
            pl.BlockSpec((1, HID), lambda i: (0, 0)),
            pl.BlockSpec((HID, D), lambda i: (0, 0)),
        ],
        out_specs=[
            pl.BlockSpec((BN, D), lambda i: (i, 0)),
            pl.BlockSpec((BN, D), lambda i: (i, 0)),
        ],
        out_shape=[
            jax.ShapeDtypeStruct((N, D), jnp.float32),
            jax.ShapeDtypeStruct((N, D), jnp.float32),
        ],
    )(acc1, x, d, W1, b1, W2)


def _tc3_body(acc_ref, q2_ref, d_ref, b2_ref, wfc_ref, bfc_ref, out_ref):
    dv = d_ref[...]
    q2 = q2_ref[...]
    h2 = jnp.maximum(
        dv * (acc_ref[0] + acc_ref[1]) + (dv * dv) * q2 + b2_ref[...], 0.0)
    out_ref[...] = (
        jnp.dot(h2, wfc_ref[...], preferred_element_type=jnp.float32)
        + bfc_ref[...])


def _tc3(acc2, q2, d, b2, Wfc, bfc):
    return pl.pallas_call(
        _tc3_body,
        grid=(N // BN,),
        in_specs=[
            pl.BlockSpec((NC, BN, D), lambda i: (0, i, 0)),
            pl.BlockSpec((BN, D), lambda i: (i, 0)),
            pl.BlockSpec((BN, 1), lambda i: (i, 0)),
            pl.BlockSpec((1, D), lambda i: (0, 0)),
            pl.BlockSpec((D, D), lambda i: (0, 0)),
            pl.BlockSpec((1, D), lambda i: (0, 0)),
        ],
        out_specs=pl.BlockSpec((BN, D), lambda i: (i, 0)),
        out_shape=jax.ShapeDtypeStruct((N, D), jnp.float32),
    )(acc2, q2, d, b2, Wfc, bfc)


def kernel(x, edge_index, W1, b1, W2, b2, Wfc, bfc):
    src = edge_index[0].reshape(NW, EW)
    dst = edge_index[1].reshape(NW, EW)
    pad = EWP - EW
    junk_s = (jnp.arange(NW, dtype=jnp.int32)[:, None] * pad
              + jnp.arange(pad, dtype=jnp.int32)[None, :] * 79) % N
    src_p = jnp.concatenate([src, junk_s], axis=1).reshape(NW, 2, HCH, CH)
    junk = (jnp.arange(NW, dtype=jnp.int32)[:, None] * pad
            + jnp.arange(pad, dtype=jnp.int32)[None, :]) % (NPAD - N) + N
    dst_p = jnp.concatenate([dst, junk], axis=1).reshape(NW, 2, HCH, CH)

    deg2 = _deg_sc(dst_p)
    xd, d = _tc1(deg2.reshape(NC, NPAD, 1), x)
    acc1 = _agg_sc(xd, src_p, dst_p)
    q2, q2d = _tc2(acc1, x, d, W1, b1.reshape(1, HID), W2)
    acc2 = _agg_sc(q2d, src_p, dst_p)
    return _tc3(acc2, q2, d, b2.reshape(1, D), Wfc, bfc.reshape(1, D))
